# baseline (speedup 1.0000x reference)
import jax
import jax.numpy as jnp
from jax.experimental import pallas as pl
from jax.experimental.pallas import tpu as pltpu

N = 10000
BM = 200
NM = N // BM


def _gcn_kernel(x_ref, adj_ref, w_ref, b_ref, out_ref, sup_ref):
    i = pl.program_id(0)

    @pl.when(i == 0)
    def _():
        sup_ref[:, :] = jnp.dot(
            x_ref[:, :], w_ref[:, :], preferred_element_type=jnp.float32
        )

    h = jnp.dot(adj_ref[:, :], sup_ref[:, :], preferred_element_type=jnp.float32)
    h = jax.nn.relu(h + b_ref[:, :])
    m = jnp.max(h, axis=1, keepdims=True)
    lse = jnp.log(jnp.sum(jnp.exp(h - m), axis=1, keepdims=True)) + m
    out_ref[:, :] = h - lse


@jax.jit
def _run(x, adj, W, b):
    nhid = W.shape[1]
    return pl.pallas_call(
        _gcn_kernel,
        grid=(NM,),
        in_specs=[
            pl.BlockSpec((N, x.shape[1]), lambda i: (0, 0)),
            pl.BlockSpec((BM, N), lambda i: (i, 0)),
            pl.BlockSpec((x.shape[1], nhid), lambda i: (0, 0)),
            pl.BlockSpec((1, nhid), lambda i: (0, 0)),
        ],
        out_specs=pl.BlockSpec((BM, nhid), lambda i: (i, 0)),
        out_shape=jax.ShapeDtypeStruct((N, nhid), jnp.float32),
        scratch_shapes=[
            pltpu.VMEM((N, nhid), jnp.float32),
        ],
    )(x, adj, W, b)


def kernel(x, adj, W, b):
    return _run(x, adj, W, b.reshape(1, -1))


# speedup vs baseline: 1.0311x; 1.0311x over previous
import jax
import jax.numpy as jnp
from jax.experimental import pallas as pl
from jax.experimental.pallas import tpu as pltpu

N = 10000
BM = 400
NM = N // BM


def _gcn_kernel(x_ref, adj_ref, w_ref, b_ref, out_ref, sup_ref):
    i = pl.program_id(0)

    @pl.when(i == 0)
    def _():
        sup_ref[:, :] = jnp.dot(
            x_ref[:, :], w_ref[:, :], preferred_element_type=jnp.float32
        )

    h = jnp.dot(adj_ref[:, :], sup_ref[:, :], preferred_element_type=jnp.float32)
    h = jax.nn.relu(h + b_ref[:, :])
    m = jnp.max(h, axis=1, keepdims=True)
    lse = jnp.log(jnp.sum(jnp.exp(h - m), axis=1, keepdims=True)) + m
    out_ref[:, :] = h - lse


@jax.jit
def _run(x, adj, W, b):
    nhid = W.shape[1]
    return pl.pallas_call(
        _gcn_kernel,
        grid=(NM,),
        in_specs=[
            pl.BlockSpec((N, x.shape[1]), lambda i: (0, 0)),
            pl.BlockSpec((BM, N), lambda i: (i, 0)),
            pl.BlockSpec((x.shape[1], nhid), lambda i: (0, 0)),
            pl.BlockSpec((1, nhid), lambda i: (0, 0)),
        ],
        out_specs=pl.BlockSpec((BM, nhid), lambda i: (i, 0)),
        out_shape=jax.ShapeDtypeStruct((N, nhid), jnp.float32),
        scratch_shapes=[
            pltpu.VMEM((N, nhid), jnp.float32),
        ],
    )(x, adj, W, b)


def kernel(x, adj, W, b):
    return _run(x, adj, W, b.reshape(1, -1))
